# baseline (speedup 1.0000x reference)
import functools

import jax
import jax.numpy as jnp
from jax import lax
from jax.experimental import pallas as pl
from jax.experimental.pallas import tpu as pltpu
from jax.experimental.pallas import tpu_sc as plsc

N = 10000
D = 128
E = 320000

NC = 2
NS = 16
NW = NC * NS
CH = 128
NCHUNK = 80
EPW = CH * NCHUNK
EP = EPW * NW
NP = 10240
RPT = NP // NS
RCH = 128
NRCH = RPT // RCH

BLK = 1024
G = NP // BLK

_mesh = plsc.VectorSubcoreMesh(
    core_axis_name="c", subcore_axis_name="s", num_cores=NC, num_subcores=NS)


@functools.partial(
    pl.kernel,
    out_type=jax.ShapeDtypeStruct((NC * NP,), jnp.float32),
    mesh=_mesh,
    scratch_types=[
        pltpu.VMEM_SHARED((NP,), jnp.float32),
        pltpu.VMEM((NCHUNK, CH), jnp.int32),
        pltpu.VMEM((CH,), jnp.float32),
        pltpu.VMEM((RPT,), jnp.float32),
    ],
)
def _sc_degree(dst_hbm, out_hbm, hist_sh, dst_v, ones_v, buf_v):
    c = lax.axis_index("c")
    s = lax.axis_index("s")
    wid = c * NS + s

    def _zero(i, _):
        buf_v[pl.ds(i * 16, 16)] = jnp.zeros((16,), jnp.float32)
        return 0

    lax.fori_loop(0, RPT // 16, _zero, 0)

    def _one(i, _):
        ones_v[pl.ds(i * 16, 16)] = jnp.ones((16,), jnp.float32)
        return 0

    lax.fori_loop(0, CH // 16, _one, 0)
    pltpu.sync_copy(buf_v, hist_sh.at[pl.ds(s * RPT, RPT)])
    pltpu.sync_copy(dst_hbm.at[pl.ds(wid * NCHUNK, NCHUNK)], dst_v)
    plsc.subcore_barrier()

    def _edges(j, _):
        pltpu.sync_copy(ones_v, hist_sh.at[dst_v.at[j]], add=True)
        return 0

    lax.fori_loop(0, NCHUNK, _edges, 0)
    plsc.subcore_barrier()

    pltpu.sync_copy(hist_sh.at[pl.ds(s * RPT, RPT)], buf_v)
    pltpu.sync_copy(buf_v, out_hbm.at[pl.ds(c * NP + s * RPT, RPT)])


NB = 2
K0 = 160
NPH0 = 4
IDXB = K0 // NPH0


@functools.partial(
    pl.kernel,
    out_type=jax.ShapeDtypeStruct((NP, D), jnp.float32),
    mesh=_mesh,
    scratch_types=[
        pltpu.VMEM_SHARED((NP, D), jnp.float32),
        pltpu.VMEM((NB, CH, D), jnp.float32),
        pltpu.VMEM((IDXB, CH), jnp.int32),
        pltpu.VMEM((IDXB, CH), jnp.int32),
        [pltpu.SemaphoreType.DMA] * NB,
    ],
)
def _sc_scatter(g_hbm, src_hbm, dst_hbm, out_hbm, acc_sh, rows_v, src_v,
                dst_v, sems):
    c = lax.axis_index("c")
    s = lax.axis_index("s")

    def _wait(b):
        pltpu.make_async_copy(g_hbm.at[pl.ds(0, CH)], rows_v.at[b],
                              sems[b]).wait()

    def _edge_pass(base_chunk, kc, nph):
        h = kc // nph
        ng = h // NB
        for p in range(nph):
            off = base_chunk + p * h
            pltpu.sync_copy(src_hbm.at[pl.ds(off, h)], src_v.at[pl.ds(0, h)])
            pltpu.sync_copy(dst_hbm.at[pl.ds(off, h)], dst_v.at[pl.ds(0, h)])
            for b in range(NB):
                pltpu.async_copy(g_hbm.at[src_v.at[b]], rows_v.at[b],
                                 sems[b])

            def _group(jg, _):
                for b in range(NB):
                    k = jg * NB + b
                    _wait(b)
                    pltpu.sync_copy(rows_v.at[b], acc_sh.at[dst_v.at[k]],
                                    add=True)
                    pltpu.async_copy(g_hbm.at[src_v.at[k + NB]],
                                     rows_v.at[b], sems[b])
                return 0

            lax.fori_loop(0, ng - 1, _group, 0)
            for b in range(NB):
                k = (ng - 1) * NB + b
                _wait(b)
                pltpu.sync_copy(rows_v.at[b], acc_sh.at[dst_v.at[k]],
                                add=True)

    @pl.when(c == 0)
    def _():
        def _zero(i, _):
            for j in range(D // 16):
                rows_v[0, i, pl.ds(j * 16, 16)] = jnp.zeros((16,),
                                                            jnp.float32)
            return 0

        lax.fori_loop(0, RCH, _zero, 0)
        for k in range(NRCH):
            pltpu.sync_copy(rows_v.at[0],
                            acc_sh.at[pl.ds(s * RPT + k * RCH, RCH)])
        plsc.subcore_barrier()

        _edge_pass(s * K0, K0, NPH0)

        plsc.subcore_barrier()

        pltpu.async_copy(acc_sh.at[pl.ds(s * RPT, RPT)],
                         out_hbm.at[pl.ds(s * RPT, RPT)],
                         sems[0]).wait()


def _dis(pid, p0, p1):
    deg = p0 + p1 + 1.0
    row = pid * BLK + lax.broadcasted_iota(jnp.int32, (BLK, 1), 0)
    return jnp.where(row < N, lax.rsqrt(deg), 0.0)


def _ln_relu_mm(u, lnw, lnb, w, dis):
    mu = jnp.mean(u, axis=1, keepdims=True)
    xc = u - mu
    var = jnp.mean(xc * xc, axis=1, keepdims=True)
    h = xc * lax.rsqrt(var + 1e-5) * lnw + lnb
    h = jnp.maximum(h, 0.0)
    return jnp.dot(h, w, preferred_element_type=jnp.float32) * dis


def _tc_first(x_ref, p0_ref, p1_ref, lnw_ref, lnb_ref, w_ref, g_ref):
    dis = _dis(pl.program_id(0), p0_ref[...], p1_ref[...])
    g_ref[...] = _ln_relu_mm(x_ref[...], lnw_ref[...], lnb_ref[...],
                             w_ref[...], dis)


def _tc_mid_res(a_ref, gp_ref, xres_ref, p0_ref, p1_ref, bp_ref,
                lnw_ref, lnb_ref, w_ref, res_ref, g_ref):
    dis = _dis(pl.program_id(0), p0_ref[...], p1_ref[...])
    u = ((a_ref[...] + gp_ref[...]) * dis + bp_ref[...]
         + xres_ref[...])
    res_ref[...] = u
    g_ref[...] = _ln_relu_mm(u, lnw_ref[...], lnb_ref[...], w_ref[...], dis)


def _tc_mid(a_ref, gp_ref, p0_ref, p1_ref, bp_ref,
            lnw_ref, lnb_ref, w_ref, g_ref):
    dis = _dis(pl.program_id(0), p0_ref[...], p1_ref[...])
    u = (a_ref[...] + gp_ref[...]) * dis + bp_ref[...]
    g_ref[...] = _ln_relu_mm(u, lnw_ref[...], lnb_ref[...], w_ref[...], dis)


def _tc_final(a_ref, gp_ref, xres_ref, p0_ref, p1_ref, bp_ref,
              out_ref):
    dis = _dis(pl.program_id(0), p0_ref[...], p1_ref[...])
    out_ref[...] = ((a_ref[...] + gp_ref[...]) * dis
                    + bp_ref[...] + xres_ref[...])


def _row_spec():
    return pl.BlockSpec((BLK, D), lambda i: (i, 0))


def _deg_specs():
    return (pl.BlockSpec((BLK, 1), lambda i: (i, 0)),
            pl.BlockSpec((BLK, 1), lambda i: (i + G, 0)))


def _vec_spec():
    return pl.BlockSpec((1, D), lambda i: (0, 0))


def _w_spec():
    return pl.BlockSpec((D, D), lambda i: (0, 0))


_cparams = pltpu.CompilerParams(dimension_semantics=("arbitrary",))

_row_shape = jax.ShapeDtypeStruct((NP, D), jnp.float32)


def _call_first(x, degp2, lnw, lnb, w):
    h0, h1 = _deg_specs()
    return pl.pallas_call(
        _tc_first,
        grid=(G,),
        in_specs=[_row_spec(), h0, h1, _vec_spec(), _vec_spec(), _w_spec()],
        out_specs=_row_spec(),
        out_shape=_row_shape,
        compiler_params=_cparams,
    )(x, degp2, degp2, lnw, lnb, w)


def _call_mid_res(part, gp, xres, degp2, bp, lnw, lnb, w):
    h0, h1 = _deg_specs()
    return pl.pallas_call(
        _tc_mid_res,
        grid=(G,),
        in_specs=[_row_spec(), _row_spec(), _row_spec(), h0, h1, _vec_spec(),
                  _vec_spec(), _vec_spec(), _w_spec()],
        out_specs=(_row_spec(), _row_spec()),
        out_shape=(_row_shape, _row_shape),
        compiler_params=_cparams,
    )(part, gp, xres, degp2, degp2, bp, lnw, lnb, w)


def _call_mid(part, gp, degp2, bp, lnw, lnb, w):
    h0, h1 = _deg_specs()
    return pl.pallas_call(
        _tc_mid,
        grid=(G,),
        in_specs=[_row_spec(), _row_spec(), h0, h1, _vec_spec(),
                  _vec_spec(), _vec_spec(), _w_spec()],
        out_specs=_row_spec(),
        out_shape=_row_shape,
        compiler_params=_cparams,
    )(part, gp, degp2, degp2, bp, lnw, lnb, w)


def _call_final(part, gp, xres, degp2, bp):
    h0, h1 = _deg_specs()
    return pl.pallas_call(
        _tc_final,
        grid=(G,),
        in_specs=[_row_spec(), _row_spec(), _row_spec(), h0, h1,
                  _vec_spec()],
        out_specs=_row_spec(),
        out_shape=_row_shape,
        compiler_params=_cparams,
    )(part, gp, xres, degp2, degp2, bp)


def kernel(x, edge_index, ln_w0, ln_b0, W0, b0, ln_w1, ln_b1, W1, b1,
           ln_w2, ln_b2, W2, b2):
    f32 = jnp.float32
    x_pad = jnp.concatenate([x.astype(f32), jnp.zeros((NP - N, D), f32)], 0)
    pad = jnp.full((EP - E,), N, jnp.int32)
    src = jnp.concatenate([edge_index[0].astype(jnp.int32), pad])
    dst = jnp.concatenate([edge_index[1].astype(jnp.int32), pad])
    src3 = src.reshape(NW * NCHUNK, CH)
    dst3 = dst.reshape(NW * NCHUNK, CH)

    degp = _sc_degree(dst3)
    degp2 = degp.reshape(2 * NP, 1)

    lnw0, lnb0 = ln_w0.reshape(1, D), ln_b0.reshape(1, D)
    lnw1, lnb1 = ln_w1.reshape(1, D), ln_b1.reshape(1, D)
    lnw2, lnb2 = ln_w2.reshape(1, D), ln_b2.reshape(1, D)
    b0r, b1r, b2r = b0.reshape(1, D), b1.reshape(1, D), b2.reshape(1, D)

    g0 = _call_first(x_pad, degp2, lnw0, lnb0, W0)
    part0 = _sc_scatter(g0, src3, dst3)
    x1, g1 = _call_mid_res(part0, g0, x_pad, degp2, b0r, lnw1, lnb1, W1)
    part1 = _sc_scatter(g1, src3, dst3)
    g2 = _call_mid(part1, g1, degp2, b1r, lnw2, lnb2, W2)
    part2 = _sc_scatter(g2, src3, dst3)
    out = _call_final(part2, g2, x1, degp2, b2r)
    return out[:N]


# speedup vs baseline: 1.2408x; 1.2408x over previous
import functools

import jax
import jax.numpy as jnp
from jax import lax
from jax.experimental import pallas as pl
from jax.experimental.pallas import tpu as pltpu
from jax.experimental.pallas import tpu_sc as plsc

N = 10000
D = 128
E = 320000

NC = 2
NS = 16
NW = NC * NS
CH = 128
NCHUNK = 80
EPW = CH * NCHUNK
EP = EPW * NW
NP = 10240
RPT = NP // NS
RCH = 128
NRCH = RPT // RCH

BLK = 1024
G = NP // BLK

_mesh = plsc.VectorSubcoreMesh(
    core_axis_name="c", subcore_axis_name="s", num_cores=NC, num_subcores=NS)


@functools.partial(
    pl.kernel,
    out_type=jax.ShapeDtypeStruct((NC * NP,), jnp.float32),
    mesh=_mesh,
    scratch_types=[
        pltpu.VMEM_SHARED((NP,), jnp.float32),
        pltpu.VMEM((NCHUNK, CH), jnp.int32),
        pltpu.VMEM((CH,), jnp.float32),
        pltpu.VMEM((RPT,), jnp.float32),
    ],
)
def _sc_degree(dst_hbm, out_hbm, hist_sh, dst_v, ones_v, buf_v):
    c = lax.axis_index("c")
    s = lax.axis_index("s")
    wid = c * NS + s

    def _zero(i, _):
        buf_v[pl.ds(i * 16, 16)] = jnp.zeros((16,), jnp.float32)
        return 0

    lax.fori_loop(0, RPT // 16, _zero, 0)

    def _one(i, _):
        ones_v[pl.ds(i * 16, 16)] = jnp.ones((16,), jnp.float32)
        return 0

    lax.fori_loop(0, CH // 16, _one, 0)
    pltpu.sync_copy(buf_v, hist_sh.at[pl.ds(s * RPT, RPT)])
    pltpu.sync_copy(dst_hbm.at[pl.ds(wid * NCHUNK, NCHUNK)], dst_v)
    plsc.subcore_barrier()

    def _edges(j, _):
        pltpu.sync_copy(ones_v, hist_sh.at[dst_v.at[j]], add=True)
        return 0

    lax.fori_loop(0, NCHUNK, _edges, 0)
    plsc.subcore_barrier()

    pltpu.sync_copy(hist_sh.at[pl.ds(s * RPT, RPT)], buf_v)
    pltpu.sync_copy(buf_v, out_hbm.at[pl.ds(c * NP + s * RPT, RPT)])


NB = 2
K0 = 128
K1 = 32
NPH0 = 4
NPH1 = 4
IDXB = K0 // NPH0


@functools.partial(
    pl.kernel,
    out_type=jax.ShapeDtypeStruct((NC * NP, D), jnp.float32),
    mesh=_mesh,
    scratch_types=[
        pltpu.VMEM_SHARED((NP, D), jnp.float32),
        pltpu.VMEM((NB, CH, D), jnp.float32),
        pltpu.VMEM((IDXB, CH), jnp.int32),
        pltpu.VMEM((IDXB, CH), jnp.int32),
        [pltpu.SemaphoreType.DMA] * NB,
    ],
)
def _sc_scatter(g_hbm, src_hbm, dst_hbm, out_hbm, acc_sh, rows_v, src_v,
                dst_v, sems):
    c = lax.axis_index("c")
    s = lax.axis_index("s")

    def _zero(i, _):
        for j in range(D // 16):
            rows_v[0, i, pl.ds(j * 16, 16)] = jnp.zeros((16,), jnp.float32)
        return 0

    lax.fori_loop(0, RCH, _zero, 0)
    for k in range(NRCH):
        pltpu.sync_copy(rows_v.at[0],
                        acc_sh.at[pl.ds(s * RPT + k * RCH, RCH)])
    plsc.subcore_barrier()

    def _wait(b):
        pltpu.make_async_copy(g_hbm.at[pl.ds(0, CH)], rows_v.at[b],
                              sems[b]).wait()

    def _edge_pass(base_chunk, kc, nph):
        h = kc // nph
        ng = h // NB
        for p in range(nph):
            off = base_chunk + p * h
            pltpu.sync_copy(src_hbm.at[pl.ds(off, h)], src_v.at[pl.ds(0, h)])
            pltpu.sync_copy(dst_hbm.at[pl.ds(off, h)], dst_v.at[pl.ds(0, h)])
            for b in range(NB):
                pltpu.async_copy(g_hbm.at[src_v.at[b]], rows_v.at[b],
                                 sems[b])

            def _group(jg, _):
                for b in range(NB):
                    k = jg * NB + b
                    _wait(b)
                    pltpu.sync_copy(rows_v.at[b], acc_sh.at[dst_v.at[k]],
                                    add=True)
                    pltpu.async_copy(g_hbm.at[src_v.at[k + NB]],
                                     rows_v.at[b], sems[b])
                return 0

            lax.fori_loop(0, ng - 1, _group, 0)
            for b in range(NB):
                k = (ng - 1) * NB + b
                _wait(b)
                pltpu.sync_copy(rows_v.at[b], acc_sh.at[dst_v.at[k]],
                                add=True)

    @pl.when(c == 0)
    def _():
        _edge_pass(s * K0, K0, NPH0)

    @pl.when(c == 1)
    def _():
        _edge_pass(NS * K0 + s * K1, K1, NPH1)

    plsc.subcore_barrier()

    pltpu.async_copy(acc_sh.at[pl.ds(s * RPT, RPT)],
                     out_hbm.at[pl.ds(c * NP + s * RPT, RPT)],
                     sems[0]).wait()


def _dis(pid, p0, p1):
    deg = p0 + p1 + 1.0
    row = pid * BLK + lax.broadcasted_iota(jnp.int32, (BLK, 1), 0)
    return jnp.where(row < N, lax.rsqrt(deg), 0.0)


def _ln_relu_mm(u, lnw, lnb, w, dis):
    mu = jnp.mean(u, axis=1, keepdims=True)
    xc = u - mu
    var = jnp.mean(xc * xc, axis=1, keepdims=True)
    h = xc * lax.rsqrt(var + 1e-5) * lnw + lnb
    h = jnp.maximum(h, 0.0)
    return jnp.dot(h, w, preferred_element_type=jnp.float32) * dis


def _tc_first(x_ref, p0_ref, p1_ref, lnw_ref, lnb_ref, w_ref, g_ref):
    dis = _dis(pl.program_id(0), p0_ref[...], p1_ref[...])
    g_ref[...] = _ln_relu_mm(x_ref[...], lnw_ref[...], lnb_ref[...],
                             w_ref[...], dis)


def _tc_mid_res(a0_ref, a1_ref, gp_ref, xres_ref, p0_ref, p1_ref, bp_ref,
                lnw_ref, lnb_ref, w_ref, res_ref, g_ref):
    dis = _dis(pl.program_id(0), p0_ref[...], p1_ref[...])
    u = ((a0_ref[...] + a1_ref[...] + gp_ref[...]) * dis + bp_ref[...]
         + xres_ref[...])
    res_ref[...] = u
    g_ref[...] = _ln_relu_mm(u, lnw_ref[...], lnb_ref[...], w_ref[...], dis)


def _tc_mid(a0_ref, a1_ref, gp_ref, p0_ref, p1_ref, bp_ref,
            lnw_ref, lnb_ref, w_ref, g_ref):
    dis = _dis(pl.program_id(0), p0_ref[...], p1_ref[...])
    u = (a0_ref[...] + a1_ref[...] + gp_ref[...]) * dis + bp_ref[...]
    g_ref[...] = _ln_relu_mm(u, lnw_ref[...], lnb_ref[...], w_ref[...], dis)


def _tc_final(a0_ref, a1_ref, gp_ref, xres_ref, p0_ref, p1_ref, bp_ref,
              out_ref):
    dis = _dis(pl.program_id(0), p0_ref[...], p1_ref[...])
    out_ref[...] = ((a0_ref[...] + a1_ref[...] + gp_ref[...]) * dis
                    + bp_ref[...] + xres_ref[...])


def _row_spec():
    return pl.BlockSpec((BLK, D), lambda i: (i, 0))


def _half_specs():
    return (pl.BlockSpec((BLK, D), lambda i: (i, 0)),
            pl.BlockSpec((BLK, D), lambda i: (i + G, 0)))


def _deg_specs():
    return (pl.BlockSpec((BLK, 1), lambda i: (i, 0)),
            pl.BlockSpec((BLK, 1), lambda i: (i + G, 0)))


def _vec_spec():
    return pl.BlockSpec((1, D), lambda i: (0, 0))


def _w_spec():
    return pl.BlockSpec((D, D), lambda i: (0, 0))


_cparams = pltpu.CompilerParams(dimension_semantics=("arbitrary",))

_row_shape = jax.ShapeDtypeStruct((NP, D), jnp.float32)


def _call_first(x, degp2, lnw, lnb, w):
    h0, h1 = _deg_specs()
    return pl.pallas_call(
        _tc_first,
        grid=(G,),
        in_specs=[_row_spec(), h0, h1, _vec_spec(), _vec_spec(), _w_spec()],
        out_specs=_row_spec(),
        out_shape=_row_shape,
        compiler_params=_cparams,
    )(x, degp2, degp2, lnw, lnb, w)


def _call_mid_res(part, gp, xres, degp2, bp, lnw, lnb, w):
    a0, a1 = _half_specs()
    h0, h1 = _deg_specs()
    return pl.pallas_call(
        _tc_mid_res,
        grid=(G,),
        in_specs=[a0, a1, _row_spec(), _row_spec(), h0, h1, _vec_spec(),
                  _vec_spec(), _vec_spec(), _w_spec()],
        out_specs=(_row_spec(), _row_spec()),
        out_shape=(_row_shape, _row_shape),
        compiler_params=_cparams,
    )(part, part, gp, xres, degp2, degp2, bp, lnw, lnb, w)


def _call_mid(part, gp, degp2, bp, lnw, lnb, w):
    a0, a1 = _half_specs()
    h0, h1 = _deg_specs()
    return pl.pallas_call(
        _tc_mid,
        grid=(G,),
        in_specs=[a0, a1, _row_spec(), h0, h1, _vec_spec(),
                  _vec_spec(), _vec_spec(), _w_spec()],
        out_specs=_row_spec(),
        out_shape=_row_shape,
        compiler_params=_cparams,
    )(part, part, gp, degp2, degp2, bp, lnw, lnb, w)


def _call_final(part, gp, xres, degp2, bp):
    a0, a1 = _half_specs()
    h0, h1 = _deg_specs()
    return pl.pallas_call(
        _tc_final,
        grid=(G,),
        in_specs=[a0, a1, _row_spec(), _row_spec(), h0, h1, _vec_spec()],
        out_specs=_row_spec(),
        out_shape=_row_shape,
        compiler_params=_cparams,
    )(part, part, gp, xres, degp2, degp2, bp)


def kernel(x, edge_index, ln_w0, ln_b0, W0, b0, ln_w1, ln_b1, W1, b1,
           ln_w2, ln_b2, W2, b2):
    f32 = jnp.float32
    x_pad = jnp.concatenate([x.astype(f32), jnp.zeros((NP - N, D), f32)], 0)
    pad = jnp.full((EP - E,), N, jnp.int32)
    src = jnp.concatenate([edge_index[0].astype(jnp.int32), pad])
    dst = jnp.concatenate([edge_index[1].astype(jnp.int32), pad])
    src3 = src.reshape(NW * NCHUNK, CH)
    dst3 = dst.reshape(NW * NCHUNK, CH)

    degp = _sc_degree(dst3)
    degp2 = degp.reshape(2 * NP, 1)

    lnw0, lnb0 = ln_w0.reshape(1, D), ln_b0.reshape(1, D)
    lnw1, lnb1 = ln_w1.reshape(1, D), ln_b1.reshape(1, D)
    lnw2, lnb2 = ln_w2.reshape(1, D), ln_b2.reshape(1, D)
    b0r, b1r, b2r = b0.reshape(1, D), b1.reshape(1, D), b2.reshape(1, D)

    g0 = _call_first(x_pad, degp2, lnw0, lnb0, W0)
    part0 = _sc_scatter(g0, src3, dst3)
    x1, g1 = _call_mid_res(part0, g0, x_pad, degp2, b0r, lnw1, lnb1, W1)
    part1 = _sc_scatter(g1, src3, dst3)
    g2 = _call_mid(part1, g1, degp2, b1r, lnw2, lnb2, W2)
    part2 = _sc_scatter(g2, src3, dst3)
    out = _call_final(part2, g2, x1, degp2, b2r)
    return out[:N]
